# Initial kernel scaffold; baseline (speedup 1.0000x reference)
import jax
import jax.numpy as jnp
from jax.experimental import pallas as pl


def kernel(combined_features, attention_mask, start_positions, end_positions, prototype_vectors, Wq, bq, Wk, bk, Wv, bv, Wo, bo, Wfc, bfc, Wqa, bqa):
    raise NotImplementedError("write your pallas kernel here")


# speedup vs baseline: 42.4646x; 42.4646x over previous
import jax
import jax.numpy as jnp
import numpy as np
from jax.experimental import pallas as pl
from jax.experimental.pallas import tpu as pltpu

B, S, H = 16, 38, 768
IMG = 512
D = H + IMG * 2
NH = 4
DK = D // NH
NP = 64

_HI = jax.lax.Precision.HIGHEST


def _dot(a, b, dims):
    return jax.lax.dot_general(a, b, (dims, ((), ())), precision=_HI,
                               preferred_element_type=jnp.float32)


def _prep_kv_kernel(proto_ref, wk_ref, bk_ref, wv_ref, bv_ref, k0_ref, v0_ref):
    p = proto_ref[...]
    k0_ref[...] = _dot(p, wk_ref[...], ((1,), (0,))) + bk_ref[...]
    v0_ref[...] = _dot(p, wv_ref[...], ((1,), (0,))) + bv_ref[...]


def _prep_t_kernel(wq_ref, bq_ref, k0_ref, t_ref, sb_ref):
    for h in range(NH):
        hs = slice(h * DK, (h + 1) * DK)
        k0h = k0_ref[:, hs]
        t_ref[:, h * NP:(h + 1) * NP] = _dot(wq_ref[:, hs], k0h, ((1,), (1,)))
        sb_ref[:, h * NP:(h + 1) * NP] = _dot(bq_ref[:, hs], k0h, ((1,), (1,)))


def _prep_w0_kernel(v0_ref, wo_ref, wfcb_ref, bo_ref, bfc_ref, w0_ref, vb_ref):
    wfcb = wfcb_ref[...]
    for h in range(NH):
        hs = slice(h * DK, (h + 1) * DK)
        u0h = _dot(v0_ref[:, hs], wo_ref[hs, :], ((1,), (0,)))
        w0_ref[h * NP:(h + 1) * NP, :] = _dot(u0h, wfcb, ((1,), (0,)))
    vb_ref[...] = _dot(bo_ref[...], wfcb, ((1,), (0,))) + bfc_ref[...]


def _main_kernel(cf_ref, t_ref, sb_ref, wfct_ref, w0_ref, vb_ref,
                 wqa_ref, bqa_ref, slog_ref, elog_ref):
    cf = cf_ref[...]
    scores = _dot(cf, t_ref[...], ((1,), (0,))) + sb_ref[...]
    n = cf.shape[0]
    iota = jax.lax.broadcasted_iota(jnp.int32, (n, NP), 1)
    wsel = jnp.zeros((n, H), dtype=jnp.float32)
    for h in range(NH):
        sh = scores[:, h * NP:(h + 1) * NP]
        m = jnp.max(sh, axis=1, keepdims=True)
        idx = jnp.min(jnp.where(sh == m, iota, NP), axis=1, keepdims=True)
        onehot = (iota == idx).astype(jnp.float32)
        wsel = wsel + _dot(onehot, w0_ref[h * NP:(h + 1) * NP, :],
                           ((1,), (0,)))
    reduced = _dot(cf, wfct_ref[...], ((1,), (0,))) + wsel + vb_ref[...]
    logits = _dot(reduced, wqa_ref[...], ((1,), (0,))) + bqa_ref[...]
    slog_ref[...] = logits[:, 0:1]
    elog_ref[...] = logits[:, 1:2]


def _loss_kernel(slog_ref, elog_ref, spos_ref, epos_ref, loss_ref):
    iota = jax.lax.broadcasted_iota(jnp.int32, (B, S), 1)

    def ce(lg, pos):
        m = jnp.max(lg, axis=1, keepdims=True)
        lse = jnp.log(jnp.sum(jnp.exp(lg - m), axis=1, keepdims=True)) + m
        sel = jnp.sum(jnp.where(iota == pos, lg, 0.0), axis=1, keepdims=True)
        return -jnp.mean(sel - lse)

    loss = 0.5 * (ce(slog_ref[...], spos_ref[...]) +
                  ce(elog_ref[...], epos_ref[...]))
    loss_ref[...] = jnp.reshape(loss, (1, 1))


def _f32(shape):
    return jax.ShapeDtypeStruct(shape, jnp.float32)


def kernel(combined_features, attention_mask, start_positions, end_positions,
           prototype_vectors, Wq, bq, Wk, bk, Wv, bv, Wo, bo, Wfc, bfc,
           Wqa, bqa):
    cf2d = combined_features.reshape(B * S, D)
    row = lambda x: x.reshape(1, -1)

    k0, v0 = pl.pallas_call(
        _prep_kv_kernel,
        out_shape=(_f32((NP, D)), _f32((NP, D))),
    )(prototype_vectors, Wk, row(bk), Wv, row(bv))

    t, sbias = pl.pallas_call(
        _prep_t_kernel,
        out_shape=(_f32((D, NH * NP)), _f32((1, NH * NP))),
    )(Wq, row(bq), k0)

    w0, vbias = pl.pallas_call(
        _prep_w0_kernel,
        out_shape=(_f32((NH * NP, H)), _f32((1, H))),
    )(v0, Wo, Wfc[D:, :], row(bo), row(bfc))

    slog, elog = pl.pallas_call(
        _main_kernel,
        out_shape=(_f32((B * S, 1)), _f32((B * S, 1))),
    )(cf2d, t, sbias, Wfc[:D, :], w0, vbias, Wqa, row(bqa))

    start_logits = slog.reshape(B, S)
    end_logits = elog.reshape(B, S)

    spos = start_positions.astype(jnp.int32).reshape(B, 1)
    epos = end_positions.astype(jnp.int32).reshape(B, 1)
    loss = pl.pallas_call(
        _loss_kernel,
        out_shape=_f32((1, 1)),
    )(start_logits, end_logits, spos, epos)[0, 0]

    return loss, start_logits, end_logits
